# Initial kernel scaffold; baseline (speedup 1.0000x reference)
import jax
import jax.numpy as jnp
from jax.experimental import pallas as pl


def kernel(obs, W1, b1, W2, b2, W3, b3, a4):
    raise NotImplementedError("write your pallas kernel here")


# speedup vs baseline: 2.7520x; 2.7520x over previous
import jax
import jax.numpy as jnp
from jax import lax
from jax.experimental import pallas as pl
from jax.experimental.pallas import tpu as pltpu

ACT = 100000


def _tc_body(obs_ref, w1_ref, b1_ref, w2_ref, b2_ref, w3t_ref, b3_ref, a4_ref,
             a_ref, logp_ref, ent_ref):
    obs = obs_ref[...]
    w1 = w1_ref[...]
    x1 = lax.dot_general(obs, w1, (((1,), (1,)), ((), ())),
                         preferred_element_type=jnp.float32)
    x1 = jnp.maximum(x1 + b1_ref[...], 0.0)
    w2 = w2_ref[...]
    x2 = lax.dot_general(x1, w2, (((1,), (1,)), ((), ())),
                         preferred_element_type=jnp.float32)
    x2 = jnp.maximum(x2 + b2_ref[...], 0.0)

    w3t = w3t_ref[...]
    z = lax.dot_general(x2, w3t, (((1,), (0,)), ((), ())),
                        preferred_element_type=jnp.float32)
    z = z + b3_ref[...]
    col = lax.broadcasted_iota(jnp.int32, (1, ACT), 1)
    a4 = a4_ref[0, 0]
    z = jnp.where(col < 2, z * a4, z)

    m = jnp.max(z)
    amax = jnp.min(jnp.where(z == m, col, ACT))
    e = jnp.exp(z - m)
    zsum = jnp.sum(e)
    za = jnp.sum(z * e)
    logz = jnp.log(zsum)
    a_ref[...] = jnp.broadcast_to(amax, (1, 1))
    logp_ref[...] = jnp.broadcast_to(-logz, (1, 1))
    ent_ref[...] = jnp.broadcast_to(m + logz - za / zsum, (1, 1))


def kernel(obs, W1, b1, W2, b2, W3, b3, a4):
    w3t = W3.T
    b3r = b3.reshape(1, ACT)
    a, logp, ent = pl.pallas_call(
        _tc_body,
        out_shape=(
            jax.ShapeDtypeStruct((1, 1), jnp.int32),
            jax.ShapeDtypeStruct((1, 1), jnp.float32),
            jax.ShapeDtypeStruct((1, 1), jnp.float32),
        ),
    )(obs, W1, b1.reshape(1, 3), W2, b2.reshape(1, 3), w3t, b3r,
      a4.reshape(1, 1))
    return (a.reshape(()), logp.reshape(1), ent.reshape(1))
